# baseline (speedup 1.0000x reference)
import functools

import jax
import jax.numpy as jnp
from jax import lax
from jax.experimental import pallas as pl
from jax.experimental.pallas import tpu as pltpu
from jax.experimental.pallas import tpu_sc as plsc

NC = 2
NS = 16
LANES = 16

EXP_CLAMP = 80.0


def _prelu(x, w):
    return jnp.where(x >= 0, x, w * x)


def _tc_qproj_body(w_ref, x_ref, wq_ref, bq_ref, wk_ref, bk_ref, wv_ref,
                   bv_ref, amat_ref, qq_ref, vq_ref, aqq_ref):
    x = x_ref[...]
    qq = jnp.dot(x, wq_ref[...], preferred_element_type=jnp.float32) + bq_ref[...]
    kq = jnp.dot(x, wk_ref[...], preferred_element_type=jnp.float32) + bk_ref[...]
    vq = jnp.dot(x, wv_ref[...], preferred_element_type=jnp.float32) + bv_ref[...]
    qq_ref[...] = qq
    vq_ref[...] = vq
    p = _prelu(qq + kq, w_ref[0, 0])
    aqq_ref[...] = jnp.dot(p, amat_ref[...], preferred_element_type=jnp.float32)


def _tc_qproj(query, wq, bq, wk, bk, wv, bv, amat, prelu_w, blk=512):
    nq, dim = query.shape
    grid = (nq // blk,)
    return pl.pallas_call(
        _tc_qproj_body,
        grid=grid,
        in_specs=[
            pl.BlockSpec((1, 1), lambda i: (0, 0), memory_space=pltpu.SMEM),
            pl.BlockSpec((blk, dim), lambda i: (i, 0)),
            pl.BlockSpec((dim, dim), lambda i: (0, 0)),
            pl.BlockSpec((1, dim), lambda i: (0, 0)),
            pl.BlockSpec((dim, dim), lambda i: (0, 0)),
            pl.BlockSpec((1, dim), lambda i: (0, 0)),
            pl.BlockSpec((dim, dim), lambda i: (0, 0)),
            pl.BlockSpec((1, dim), lambda i: (0, 0)),
            pl.BlockSpec((dim, 16), lambda i: (0, 0)),
        ],
        out_specs=[
            pl.BlockSpec((blk, dim), lambda i: (i, 0)),
            pl.BlockSpec((blk, dim), lambda i: (i, 0)),
            pl.BlockSpec((blk, 16), lambda i: (i, 0)),
        ],
        out_shape=[
            jax.ShapeDtypeStruct((nq, dim), jnp.float32),
            jax.ShapeDtypeStruct((nq, dim), jnp.float32),
            jax.ShapeDtypeStruct((nq, 16), jnp.float32),
        ],
    )(prelu_w.reshape(1, 1), query, wq, bq.reshape(1, dim), wk,
      bk.reshape(1, dim), wv, bv.reshape(1, dim), amat)


def _tc_kall_body(k_ref, wk_ref, bk_ref, kall_ref):
    kall_ref[...] = (jnp.dot(k_ref[...], wk_ref[...],
                             preferred_element_type=jnp.float32) + bk_ref[...])


def _tc_kall(k0, wk, bk, blk=512):
    nk, dim = k0.shape
    grid = (nk // blk,)
    return pl.pallas_call(
        _tc_kall_body,
        grid=grid,
        in_specs=[
            pl.BlockSpec((blk, dim), lambda i: (i, 0)),
            pl.BlockSpec((dim, dim), lambda i: (0, 0)),
            pl.BlockSpec((1, dim), lambda i: (0, 0)),
        ],
        out_specs=pl.BlockSpec((blk, dim), lambda i: (i, 0)),
        out_shape=jax.ShapeDtypeStruct((nk, dim), jnp.float32),
    )(k0, wk, bk.reshape(1, dim))


def _tc_vhead_body(v_ref, wv_ref, bv_ref, vh_ref, *, h, dph):
    vall = (jnp.dot(v_ref[...], wv_ref[...],
                    preferred_element_type=jnp.float32) + bv_ref[...])
    for hh in range(h):
        vh_ref[hh] = vall[:, hh * dph:(hh + 1) * dph]


def _tc_vhead(v0, wv, bv, h, dph, blk=512):
    nk, dim = v0.shape
    grid = (nk // blk,)
    return pl.pallas_call(
        functools.partial(_tc_vhead_body, h=h, dph=dph),
        grid=grid,
        in_specs=[
            pl.BlockSpec((blk, dim), lambda i: (i, 0)),
            pl.BlockSpec((dim, dim), lambda i: (0, 0)),
            pl.BlockSpec((1, dim), lambda i: (0, 0)),
        ],
        out_specs=pl.BlockSpec((h, blk, dph), lambda i: (0, i, 0)),
        out_shape=jax.ShapeDtypeStruct((h, nk, dph), jnp.float32),
    )(v0, wv, bv.reshape(1, dim))


def _sc_stage1(qi, ki, qq, kall, aqq, apw, nheads, dph):
    e = qi.shape[0]
    nq, dim = qq.shape
    chunk = 32
    nchunks = e // (NC * NS * chunk)
    qi2 = qi.reshape(e // chunk, chunk)
    ki2 = ki.reshape(e // chunk, chunk)

    mesh = plsc.VectorSubcoreMesh(core_axis_name="c", subcore_axis_name="s")

    @functools.partial(
        pl.kernel,
        out_type=[
            jax.ShapeDtypeStruct((nheads, e), jnp.float32),
            jax.ShapeDtypeStruct((NC, nq, 16), jnp.float32),
        ],
        mesh=mesh,
        scratch_types=[
            pltpu.VMEM((e // (NC * NS * chunk), chunk), jnp.int32),
            pltpu.VMEM((e // (NC * NS * chunk), chunk), jnp.int32),
            pltpu.VMEM((chunk, 512), jnp.float32),
            pltpu.VMEM((chunk, 512), jnp.float32),
            pltpu.VMEM((chunk, 512), jnp.float32),
            pltpu.VMEM((chunk, 512), jnp.float32),
            pltpu.VMEM((chunk, 16), jnp.float32),
            pltpu.VMEM((chunk, 16), jnp.float32),
            pltpu.VMEM((512 + LANES,), jnp.float32),
            pltpu.VMEM((chunk, 16), jnp.float32),
            pltpu.VMEM((nheads, e // (NC * NS)), jnp.float32),
            pltpu.VMEM_SHARED((nq, 16), jnp.float32),
            pltpu.SemaphoreType.DMA,
            pltpu.SemaphoreType.DMA,
        ],
        compiler_params=pltpu.CompilerParams(use_tc_tiling_on_sc=False, needs_layout_passes=False),
    )
    def stage1(qi_hbm, ki_hbm, qq_hbm, kall_hbm, aqq_hbm, apw_hbm, w_hbm,
               den_hbm, qi_t, ki_t, qrows0, qrows1, krows0, krows1, arows0,
               arows1, a_v, wrow, whead, den_sp, sem0, sem1):
        cid = lax.axis_index("c")
        sid = lax.axis_index("s")
        wid = cid * NS + sid
        row0 = wid * nchunks
        ebase = row0 * chunk

        pltpu.sync_copy(apw_hbm, a_v)
        pw = a_v[pl.ds(dim, LANES)]

        zrows = nq // NS

        def zrow(j, carry):
            wrow[j, pl.ds(0, LANES)] = jnp.zeros((LANES,), jnp.float32)
            return carry

        lax.fori_loop(0, chunk, zrow, 0)
        for r0 in range(0, zrows, chunk):
            pltpu.sync_copy(wrow, den_sp.at[pl.ds(sid * zrows + r0, chunk)])
        plsc.subcore_barrier()

        pltpu.sync_copy(qi_hbm.at[pl.ds(row0, nchunks)], qi_t)
        pltpu.sync_copy(ki_hbm.at[pl.ds(row0, nchunks)], ki_t)

        qbufs = (qrows0, qrows1)
        kbufs = (krows0, krows1)
        abufs = (arows0, arows1)
        sems = (sem0, sem1)

        def gather(ci, b):
            pltpu.async_copy(qq_hbm.at[qi_t.at[ci]], qbufs[b], sems[b])
            pltpu.async_copy(kall_hbm.at[ki_t.at[ci]], kbufs[b], sems[b])
            pltpu.async_copy(aqq_hbm.at[qi_t.at[ci]], abufs[b], sems[b])

        def gather_wait(ci, b):
            pltpu.make_async_copy(qq_hbm.at[qi_t.at[ci]], qbufs[b],
                                  sems[b]).wait()
            pltpu.make_async_copy(kall_hbm.at[ki_t.at[ci]], kbufs[b],
                                  sems[b]).wait()
            pltpu.make_async_copy(aqq_hbm.at[qi_t.at[ci]], abufs[b],
                                  sems[b]).wait()

        gather(0, 0)
        gather(1, 1)

        lane = lax.iota(jnp.int32, LANES)

        def do_chunk(o, carry):
            for b in range(2):
                ci = 2 * o + b
                gather_wait(ci, b)
                qrows, krows, arows = qbufs[b], kbufs[b], abufs[b]

                def do_edge(j, carry2):
                    aq = arows[j, pl.ds(0, LANES)]
                    v = jnp.zeros((LANES,), jnp.float32)
                    for hh in range(nheads):
                        acc = jnp.zeros((LANES,), jnp.float32)
                        for c4 in range(dph // LANES):
                            d0 = hh * dph + c4 * LANES
                            s = (qrows[j, pl.ds(d0, LANES)]
                                 + krows[j, pl.ds(d0, LANES)])
                            p = jnp.where(s >= 0, s, pw * s)
                            acc = acc + a_v[pl.ds(d0, LANES)] * p
                        for sh in (8, 4, 2, 1):
                            acc = acc + lax.gather(
                                acc, ((lane + sh) & (LANES - 1))[:, None],
                                lax.GatherDimensionNumbers(
                                    offset_dims=(),
                                    collapsed_slice_dims=(0,),
                                    start_index_map=(0,)),
                                (1,),
                                mode=lax.GatherScatterMode.PROMISE_IN_BOUNDS)
                        v = jnp.where(lane == hh, acc, v)
                    v = jnp.where(lane < nheads,
                                  jnp.exp(jnp.minimum(v - aq, EXP_CLAMP)),
                                  0.0)
                    wrow[j, pl.ds(0, LANES)] = v
                    return carry2

                lax.fori_loop(0, chunk, do_edge, 0)

                for hh in range(nheads):
                    col_h = jnp.full((LANES,), hh, jnp.int32)
                    for g in range(chunk // LANES):
                        rows = lane + g * LANES
                        w16 = plsc.load_gather(wrow, [rows, col_h])
                        whead[hh, pl.ds(ci * chunk + g * LANES, LANES)] = w16

                pltpu.sync_copy(wrow, den_sp.at[qi_t.at[ci]], add=True)

                @pl.when(ci + 2 < nchunks)
                def _():
                    gather(ci + 2, b)

            return carry

        lax.fori_loop(0, nchunks // 2, do_chunk, 0)

        for hh in range(nheads):
            pltpu.sync_copy(whead.at[hh],
                            w_hbm.at[hh, pl.ds(ebase, nchunks * chunk)])

        plsc.subcore_barrier()
        for r0 in range(0, zrows, chunk):
            r = sid * zrows + r0
            pltpu.sync_copy(den_sp.at[pl.ds(r, chunk)],
                            den_hbm.at[cid, pl.ds(r, chunk)])

    return stage1(qi2, ki2, qq, kall, aqq, apw)


def _sc_stage2(qi, vi, wexp, vhead, nq, nheads, dph):
    e = qi.shape[0]
    chunk = 128
    nchunks = e // (NS * chunk)
    hpc = nheads // NC
    qi2 = qi.reshape(e // chunk, chunk)
    vi2 = vi.reshape(e // chunk, chunk)
    w3 = wexp.reshape(nheads, e // chunk, chunk)

    mesh = plsc.VectorSubcoreMesh(core_axis_name="c", subcore_axis_name="s")

    @functools.partial(
        pl.kernel,
        out_type=jax.ShapeDtypeStruct((nheads, nq, dph), jnp.bfloat16),
        mesh=mesh,
        scratch_types=[
            pltpu.VMEM((nchunks, chunk), jnp.int32),
            pltpu.VMEM((nchunks, chunk), jnp.int32),
            pltpu.VMEM((nchunks, chunk), jnp.float32),
            pltpu.VMEM((chunk, dph), jnp.float32),
            pltpu.VMEM((chunk, dph), jnp.float32),
            pltpu.VMEM((chunk, dph), jnp.bfloat16),
            pltpu.VMEM((chunk, dph), jnp.bfloat16),
            pltpu.VMEM_SHARED((nq, dph), jnp.bfloat16),
            pltpu.SemaphoreType.DMA,
            pltpu.SemaphoreType.DMA,
        ],
        compiler_params=pltpu.CompilerParams(use_tc_tiling_on_sc=False, needs_layout_passes=False),
    )
    def stage2(qi_hbm, vi_hbm, w_hbm, vh_hbm, msg_hbm, qi_t, vi_t, w_t,
               vrows0, vrows1, brows, zbuf, acc_sp, sem0, sem1):
        cid = lax.axis_index("c")
        sid = lax.axis_index("s")
        zrows = nq // NS
        row0 = sid * nchunks

        def zrow(j, carry):
            for c4 in range(0, dph, 2 * LANES):
                zbuf[j, pl.ds(c4, 2 * LANES)] = jnp.zeros((2 * LANES,),
                                                          jnp.bfloat16)
            return carry

        lax.fori_loop(0, chunk, zrow, 0)
        pltpu.sync_copy(qi_hbm.at[pl.ds(row0, nchunks)], qi_t)
        pltpu.sync_copy(vi_hbm.at[pl.ds(row0, nchunks)], vi_t)

        vbufs = (vrows0, vrows1)
        sems = (sem0, sem1)

        def gather(ci, b):
            pltpu.async_copy(vh_hbm.at[hd[0]].at[vi_t.at[ci]],
                             vbufs[b], sems[b])

        def gather_wait(ci, b):
            pltpu.make_async_copy(vh_hbm.at[hd[0]].at[vi_t.at[ci]],
                                  vbufs[b], sems[b]).wait()

        hd = [0]
        for hh in range(hpc):
            head = cid * hpc + hh
            hd[0] = head
            for r0 in range(0, zrows, chunk):
                pltpu.sync_copy(zbuf,
                                acc_sp.at[pl.ds(sid * zrows + r0, chunk)])
            plsc.subcore_barrier()
            pltpu.sync_copy(w_hbm.at[head, pl.ds(row0, nchunks)], w_t)

            gather(0, 0)
            gather(1, 1)

            def do_chunk(o, carry):
                for b in range(2):
                    ci = 2 * o + b
                    gather_wait(ci, b)

                    def do_group(g, carry2):
                        w16 = w_t[ci, pl.ds(g * LANES, LANES)]
                        vrows = vbufs[b]
                        for jj in range(LANES):
                            w = w16[jj]
                            row = g * LANES + jj
                            for c4 in range(dph // (2 * LANES)):
                                d0 = c4 * 2 * LANES
                                lo = vrows[row, pl.ds(d0, LANES)] * w
                                hi = vrows[row, pl.ds(d0 + LANES, LANES)] * w
                                brows[row, pl.ds(d0, 2 * LANES)] = plsc.pack(
                                    lo, hi,
                                    format=plsc.PackFormat.INTERLEAVED)
                        return carry2

                    lax.fori_loop(0, chunk // LANES, do_group, 0)
                    pltpu.sync_copy(brows, acc_sp.at[qi_t.at[ci]], add=True)

                    @pl.when(ci + 2 < nchunks)
                    def _():
                        gather(ci + 2, b)

                return carry

            lax.fori_loop(0, nchunks // 2, do_chunk, 0)

            plsc.subcore_barrier()
            for r0 in range(0, zrows, chunk):
                r = sid * zrows + r0
                pltpu.sync_copy(acc_sp.at[pl.ds(r, chunk)],
                                msg_hbm.at[head, pl.ds(r, chunk)])
            plsc.subcore_barrier()

    return stage2(qi2, vi2, w3, vhead)


def _tc_final_body(x_ref, vq_ref, msg_ref, den_ref, wp_ref, bp_ref, g_ref,
                   b_ref, out_ref, *, h, dph):
    den = 1.0 + den_ref[0] + den_ref[1]
    parts = []
    for hh in range(h):
        m = (msg_ref[hh].astype(jnp.float32)
             + vq_ref[:, hh * dph:(hh + 1) * dph])
        parts.append(m / den[:, hh:hh + 1])
    msg = jnp.concatenate(parts, axis=-1)
    y = jnp.dot(msg, wp_ref[...], preferred_element_type=jnp.float32) + bp_ref[...]
    r = x_ref[...] + y
    mu = jnp.mean(r, axis=-1, keepdims=True)
    var = jnp.mean((r - mu) ** 2, axis=-1, keepdims=True)
    out_ref[...] = (r - mu) / jnp.sqrt(var + 1e-5) * g_ref[...] + b_ref[...]


def _tc_final(query, vq, msg, den, wp, bp, ln_g, ln_b, h, dph, blk=512):
    nq, dim = query.shape
    grid = (nq // blk,)
    return pl.pallas_call(
        functools.partial(_tc_final_body, h=h, dph=dph),
        grid=grid,
        in_specs=[
            pl.BlockSpec((blk, dim), lambda i: (i, 0)),
            pl.BlockSpec((blk, dim), lambda i: (i, 0)),
            pl.BlockSpec((h, blk, dph), lambda i: (0, i, 0)),
            pl.BlockSpec((2, blk, 16), lambda i: (0, i, 0)),
            pl.BlockSpec((dim, dim), lambda i: (0, 0)),
            pl.BlockSpec((1, dim), lambda i: (0, 0)),
            pl.BlockSpec((1, dim), lambda i: (0, 0)),
            pl.BlockSpec((1, dim), lambda i: (0, 0)),
        ],
        out_specs=pl.BlockSpec((blk, dim), lambda i: (i, 0)),
        out_shape=jax.ShapeDtypeStruct((nq, dim), jnp.float32),
    )(query, vq, msg, den, wp, bp.reshape(1, dim), ln_g.reshape(1, dim),
      ln_b.reshape(1, dim))


def kernel(query, keys, values, query_len, self_attn, query_idxs, key_idxs,
           value_idxs, Wq, bq, Wk, bk, Wv, bv, Wp, bp, a, prelu_w, ln_g,
           ln_b):
    k0 = keys[0]
    v0 = values[0]
    qi = query_idxs[0].astype(jnp.int32)
    ki = key_idxs[0].astype(jnp.int32)
    vi = value_idxs[0].astype(jnp.int32)

    nq, dim = query.shape
    h = a.shape[1]
    dph = a.shape[2]

    amat = jnp.zeros((dim, 16), jnp.float32)
    for hh in range(h):
        amat = amat.at[hh * dph:(hh + 1) * dph, hh].set(a[0, hh])

    qq, vq, aqq = _tc_qproj(query, Wq, bq, Wk, bk, Wv, bv, amat, prelu_w)
    kall = _tc_kall(k0, Wk, bk)
    perm32 = jnp.concatenate(
        [jnp.arange(0, 32, 2), jnp.arange(1, 32, 2)])
    perm = (jnp.arange(dim) // 32) * 32 + perm32[jnp.arange(dim) % 32]
    vhead = _tc_vhead(v0, Wv[:, perm], bv[perm], h, dph)
    apw = jnp.concatenate(
        [a.reshape(dim), jnp.full((LANES,), prelu_w, jnp.float32)])
    wexp, den = _sc_stage1(qi, ki, qq, kall, aqq, apw, h, dph)
    msg = _sc_stage2(qi, vi, wexp, vhead, nq, h, dph)
    return _tc_final(query, vq, msg, den, Wp, bp, ln_g, ln_b, h, dph)


# speedup vs baseline: 1.0873x; 1.0873x over previous
import functools

import jax
import jax.numpy as jnp
from jax import lax
from jax.experimental import pallas as pl
from jax.experimental.pallas import tpu as pltpu
from jax.experimental.pallas import tpu_sc as plsc

NC = 2
NS = 16
LANES = 16

EXP_CLAMP = 80.0


def _prelu(x, w):
    return jnp.where(x >= 0, x, w * x)


def _tc_qproj_body(w_ref, x_ref, wq_ref, bq_ref, wk_ref, bk_ref, wv_ref,
                   bv_ref, amat_ref, qq_ref, vq_ref, aqq_ref):
    x = x_ref[...].astype(jnp.bfloat16)
    qq = (jnp.dot(x, wq_ref[...].astype(jnp.bfloat16),
                  preferred_element_type=jnp.float32) + bq_ref[...])
    kq = (jnp.dot(x, wk_ref[...].astype(jnp.bfloat16),
                  preferred_element_type=jnp.float32) + bk_ref[...])
    vq = (jnp.dot(x, wv_ref[...].astype(jnp.bfloat16),
                  preferred_element_type=jnp.float32) + bv_ref[...])
    qq_ref[...] = qq
    vq_ref[...] = vq
    p = _prelu(qq + kq, w_ref[0, 0]).astype(jnp.bfloat16)
    aqq_ref[...] = jnp.dot(p, amat_ref[...].astype(jnp.bfloat16),
                           preferred_element_type=jnp.float32)


def _tc_qproj(query, wq, bq, wk, bk, wv, bv, amat, prelu_w, blk=512):
    nq, dim = query.shape
    grid = (nq // blk,)
    return pl.pallas_call(
        _tc_qproj_body,
        grid=grid,
        in_specs=[
            pl.BlockSpec((1, 1), lambda i: (0, 0), memory_space=pltpu.SMEM),
            pl.BlockSpec((blk, dim), lambda i: (i, 0)),
            pl.BlockSpec((dim, dim), lambda i: (0, 0)),
            pl.BlockSpec((1, dim), lambda i: (0, 0)),
            pl.BlockSpec((dim, dim), lambda i: (0, 0)),
            pl.BlockSpec((1, dim), lambda i: (0, 0)),
            pl.BlockSpec((dim, dim), lambda i: (0, 0)),
            pl.BlockSpec((1, dim), lambda i: (0, 0)),
            pl.BlockSpec((dim, 16), lambda i: (0, 0)),
        ],
        out_specs=[
            pl.BlockSpec((blk, dim), lambda i: (i, 0)),
            pl.BlockSpec((blk, dim), lambda i: (i, 0)),
            pl.BlockSpec((blk, 16), lambda i: (i, 0)),
        ],
        out_shape=[
            jax.ShapeDtypeStruct((nq, dim), jnp.float32),
            jax.ShapeDtypeStruct((nq, dim), jnp.float32),
            jax.ShapeDtypeStruct((nq, 16), jnp.float32),
        ],
    )(prelu_w.reshape(1, 1), query, wq, bq.reshape(1, dim), wk,
      bk.reshape(1, dim), wv, bv.reshape(1, dim), amat)


def _tc_kv_body(k_ref, v_ref, wk_ref, bk_ref, wv_ref, bv_ref, kall_ref,
                vh_ref, *, h, dph):
    kb = k_ref[...].astype(jnp.bfloat16)
    vb = v_ref[...].astype(jnp.bfloat16)
    kall_ref[...] = (jnp.dot(kb, wk_ref[...].astype(jnp.bfloat16),
                             preferred_element_type=jnp.float32) + bk_ref[...])
    vall = (jnp.dot(vb, wv_ref[...].astype(jnp.bfloat16),
                    preferred_element_type=jnp.float32) + bv_ref[...])
    for hh in range(h):
        vh_ref[hh] = vall[:, hh * dph:(hh + 1) * dph]


def _tc_kv(k0, v0, wk, bk, wv, bv, h, dph, blk=512):
    nk, dim = k0.shape
    grid = (nk // blk,)
    return pl.pallas_call(
        functools.partial(_tc_kv_body, h=h, dph=dph),
        grid=grid,
        in_specs=[
            pl.BlockSpec((blk, dim), lambda i: (i, 0)),
            pl.BlockSpec((blk, dim), lambda i: (i, 0)),
            pl.BlockSpec((dim, dim), lambda i: (0, 0)),
            pl.BlockSpec((1, dim), lambda i: (0, 0)),
            pl.BlockSpec((dim, dim), lambda i: (0, 0)),
            pl.BlockSpec((1, dim), lambda i: (0, 0)),
        ],
        out_specs=[
            pl.BlockSpec((blk, dim), lambda i: (i, 0)),
            pl.BlockSpec((h, blk, dph), lambda i: (0, i, 0)),
        ],
        out_shape=[
            jax.ShapeDtypeStruct((nk, dim), jnp.float32),
            jax.ShapeDtypeStruct((h, nk, dph), jnp.float32),
        ],
    )(k0, v0, wk, bk.reshape(1, dim), wv, bv.reshape(1, dim))


def _sc_stage1(qi, ki, qq, kall, aqq, apw, nheads, dph):
    e = qi.shape[0]
    nq, dim = qq.shape
    chunk = 32
    nchunks = e // (NC * NS * chunk)
    qi2 = qi.reshape(e // chunk, chunk)
    ki2 = ki.reshape(e // chunk, chunk)

    mesh = plsc.VectorSubcoreMesh(core_axis_name="c", subcore_axis_name="s")

    @functools.partial(
        pl.kernel,
        out_type=[
            jax.ShapeDtypeStruct((nheads, e), jnp.float32),
            jax.ShapeDtypeStruct((NC, nq, 16), jnp.float32),
        ],
        mesh=mesh,
        scratch_types=[
            pltpu.VMEM((e // (NC * NS * chunk), chunk), jnp.int32),
            pltpu.VMEM((e // (NC * NS * chunk), chunk), jnp.int32),
            pltpu.VMEM((chunk, 512), jnp.float32),
            pltpu.VMEM((chunk, 512), jnp.float32),
            pltpu.VMEM((chunk, 512), jnp.float32),
            pltpu.VMEM((chunk, 512), jnp.float32),
            pltpu.VMEM((chunk, 16), jnp.float32),
            pltpu.VMEM((chunk, 16), jnp.float32),
            pltpu.VMEM((512 + LANES,), jnp.float32),
            pltpu.VMEM((chunk, 16), jnp.float32),
            pltpu.VMEM((nheads, e // (NC * NS)), jnp.float32),
            pltpu.VMEM_SHARED((nq, 16), jnp.float32),
            pltpu.SemaphoreType.DMA,
            pltpu.SemaphoreType.DMA,
        ],
        compiler_params=pltpu.CompilerParams(use_tc_tiling_on_sc=False, needs_layout_passes=False),
    )
    def stage1(qi_hbm, ki_hbm, qq_hbm, kall_hbm, aqq_hbm, apw_hbm, w_hbm,
               den_hbm, qi_t, ki_t, qrows0, qrows1, krows0, krows1, arows0,
               arows1, a_v, wrow, whead, den_sp, sem0, sem1):
        cid = lax.axis_index("c")
        sid = lax.axis_index("s")
        wid = cid * NS + sid
        row0 = wid * nchunks
        ebase = row0 * chunk

        pltpu.sync_copy(apw_hbm, a_v)
        pw = a_v[pl.ds(dim, LANES)]

        zrows = nq // NS

        def zrow(j, carry):
            wrow[j, pl.ds(0, LANES)] = jnp.zeros((LANES,), jnp.float32)
            return carry

        lax.fori_loop(0, chunk, zrow, 0)
        for r0 in range(0, zrows, chunk):
            pltpu.sync_copy(wrow, den_sp.at[pl.ds(sid * zrows + r0, chunk)])
        plsc.subcore_barrier()

        pltpu.sync_copy(qi_hbm.at[pl.ds(row0, nchunks)], qi_t)
        pltpu.sync_copy(ki_hbm.at[pl.ds(row0, nchunks)], ki_t)

        qbufs = (qrows0, qrows1)
        kbufs = (krows0, krows1)
        abufs = (arows0, arows1)
        sems = (sem0, sem1)

        def gather(ci, b):
            pltpu.async_copy(qq_hbm.at[qi_t.at[ci]], qbufs[b], sems[b])
            pltpu.async_copy(kall_hbm.at[ki_t.at[ci]], kbufs[b], sems[b])
            pltpu.async_copy(aqq_hbm.at[qi_t.at[ci]], abufs[b], sems[b])

        def gather_wait(ci, b):
            pltpu.make_async_copy(qq_hbm.at[qi_t.at[ci]], qbufs[b],
                                  sems[b]).wait()
            pltpu.make_async_copy(kall_hbm.at[ki_t.at[ci]], kbufs[b],
                                  sems[b]).wait()
            pltpu.make_async_copy(aqq_hbm.at[qi_t.at[ci]], abufs[b],
                                  sems[b]).wait()

        gather(0, 0)
        gather(1, 1)

        lane = lax.iota(jnp.int32, LANES)

        def do_chunk(o, carry):
            for b in range(2):
                ci = 2 * o + b
                gather_wait(ci, b)
                qrows, krows, arows = qbufs[b], kbufs[b], abufs[b]

                def do_edge(j, carry2):
                    aq = arows[j, pl.ds(0, LANES)]
                    v = jnp.zeros((LANES,), jnp.float32)
                    for hh in range(nheads):
                        acc = jnp.zeros((LANES,), jnp.float32)
                        for c4 in range(dph // LANES):
                            d0 = hh * dph + c4 * LANES
                            s = (qrows[j, pl.ds(d0, LANES)]
                                 + krows[j, pl.ds(d0, LANES)])
                            p = jnp.where(s >= 0, s, pw * s)
                            acc = acc + a_v[pl.ds(d0, LANES)] * p
                        for sh in (8, 4, 2, 1):
                            acc = acc + lax.gather(
                                acc, ((lane + sh) & (LANES - 1))[:, None],
                                lax.GatherDimensionNumbers(
                                    offset_dims=(),
                                    collapsed_slice_dims=(0,),
                                    start_index_map=(0,)),
                                (1,),
                                mode=lax.GatherScatterMode.PROMISE_IN_BOUNDS)
                        v = jnp.where(lane == hh, acc, v)
                    v = jnp.where(lane < nheads,
                                  jnp.exp(jnp.minimum(v - aq, EXP_CLAMP)),
                                  0.0)
                    wrow[j, pl.ds(0, LANES)] = v
                    return carry2

                lax.fori_loop(0, chunk, do_edge, 0)

                for hh in range(nheads):
                    col_h = jnp.full((LANES,), hh, jnp.int32)
                    for g in range(chunk // LANES):
                        rows = lane + g * LANES
                        w16 = plsc.load_gather(wrow, [rows, col_h])
                        whead[hh, pl.ds(ci * chunk + g * LANES, LANES)] = w16

                pltpu.sync_copy(wrow, den_sp.at[qi_t.at[ci]], add=True)

                @pl.when(ci + 2 < nchunks)
                def _():
                    gather(ci + 2, b)

            return carry

        lax.fori_loop(0, nchunks // 2, do_chunk, 0)

        for hh in range(nheads):
            pltpu.sync_copy(whead.at[hh],
                            w_hbm.at[hh, pl.ds(ebase, nchunks * chunk)])

        plsc.subcore_barrier()
        for r0 in range(0, zrows, chunk):
            r = sid * zrows + r0
            pltpu.sync_copy(den_sp.at[pl.ds(r, chunk)],
                            den_hbm.at[cid, pl.ds(r, chunk)])

    return stage1(qi2, ki2, qq, kall, aqq, apw)


def _sc_stage2(qi, vi, wexp, vhead, nq, nheads, dph):
    e = qi.shape[0]
    chunk = 128
    nchunks = e // (NS * chunk)
    hpc = nheads // NC
    qi2 = qi.reshape(e // chunk, chunk)
    vi2 = vi.reshape(e // chunk, chunk)
    w3 = wexp.reshape(nheads, e // chunk, chunk)

    mesh = plsc.VectorSubcoreMesh(core_axis_name="c", subcore_axis_name="s")

    @functools.partial(
        pl.kernel,
        out_type=jax.ShapeDtypeStruct((nheads, nq, dph), jnp.bfloat16),
        mesh=mesh,
        scratch_types=[
            pltpu.VMEM((nchunks, chunk), jnp.int32),
            pltpu.VMEM((nchunks, chunk), jnp.int32),
            pltpu.VMEM((nchunks, chunk), jnp.float32),
            pltpu.VMEM((chunk, dph), jnp.float32),
            pltpu.VMEM((chunk, dph), jnp.float32),
            pltpu.VMEM((chunk, dph), jnp.bfloat16),
            pltpu.VMEM((chunk, dph), jnp.bfloat16),
            pltpu.VMEM_SHARED((nq, dph), jnp.bfloat16),
            pltpu.SemaphoreType.DMA,
            pltpu.SemaphoreType.DMA,
        ],
        compiler_params=pltpu.CompilerParams(use_tc_tiling_on_sc=False, needs_layout_passes=False),
    )
    def stage2(qi_hbm, vi_hbm, w_hbm, vh_hbm, msg_hbm, qi_t, vi_t, w_t,
               vrows0, vrows1, brows, zbuf, acc_sp, sem0, sem1):
        cid = lax.axis_index("c")
        sid = lax.axis_index("s")
        zrows = nq // NS
        row0 = sid * nchunks

        def zrow(j, carry):
            for c4 in range(0, dph, 2 * LANES):
                zbuf[j, pl.ds(c4, 2 * LANES)] = jnp.zeros((2 * LANES,),
                                                          jnp.bfloat16)
            return carry

        lax.fori_loop(0, chunk, zrow, 0)
        pltpu.sync_copy(qi_hbm.at[pl.ds(row0, nchunks)], qi_t)
        pltpu.sync_copy(vi_hbm.at[pl.ds(row0, nchunks)], vi_t)

        vbufs = (vrows0, vrows1)
        sems = (sem0, sem1)

        def gather(ci, b):
            pltpu.async_copy(vh_hbm.at[hd[0]].at[vi_t.at[ci]],
                             vbufs[b], sems[b])

        def gather_wait(ci, b):
            pltpu.make_async_copy(vh_hbm.at[hd[0]].at[vi_t.at[ci]],
                                  vbufs[b], sems[b]).wait()

        hd = [0]
        for hh in range(hpc):
            head = cid * hpc + hh
            hd[0] = head
            for r0 in range(0, zrows, chunk):
                pltpu.sync_copy(zbuf,
                                acc_sp.at[pl.ds(sid * zrows + r0, chunk)])
            plsc.subcore_barrier()
            pltpu.sync_copy(w_hbm.at[head, pl.ds(row0, nchunks)], w_t)

            gather(0, 0)
            gather(1, 1)

            def do_chunk(o, carry):
                for b in range(2):
                    ci = 2 * o + b
                    gather_wait(ci, b)

                    def do_group(g, carry2):
                        w16 = w_t[ci, pl.ds(g * LANES, LANES)]
                        vrows = vbufs[b]
                        for jj in range(LANES):
                            w = w16[jj]
                            row = g * LANES + jj
                            for c4 in range(dph // (2 * LANES)):
                                d0 = c4 * 2 * LANES
                                lo = vrows[row, pl.ds(d0, LANES)] * w
                                hi = vrows[row, pl.ds(d0 + LANES, LANES)] * w
                                brows[row, pl.ds(d0, 2 * LANES)] = plsc.pack(
                                    lo, hi,
                                    format=plsc.PackFormat.INTERLEAVED)
                        return carry2

                    lax.fori_loop(0, chunk // LANES, do_group, 0)
                    pltpu.sync_copy(brows, acc_sp.at[qi_t.at[ci]], add=True)

                    @pl.when(ci + 2 < nchunks)
                    def _():
                        gather(ci + 2, b)

                return carry

            lax.fori_loop(0, nchunks // 2, do_chunk, 0)

            plsc.subcore_barrier()
            for r0 in range(0, zrows, chunk):
                r = sid * zrows + r0
                pltpu.sync_copy(acc_sp.at[pl.ds(r, chunk)],
                                msg_hbm.at[head, pl.ds(r, chunk)])
            plsc.subcore_barrier()

    return stage2(qi2, vi2, w3, vhead)


def _tc_final_body(x_ref, vq_ref, msg_ref, den_ref, wp_ref, bp_ref, g_ref,
                   b_ref, out_ref, *, h, dph):
    den = 1.0 + den_ref[0] + den_ref[1]
    parts = []
    for hh in range(h):
        m = (msg_ref[hh].astype(jnp.float32)
             + vq_ref[:, hh * dph:(hh + 1) * dph])
        parts.append(m / den[:, hh:hh + 1])
    msg = jnp.concatenate(parts, axis=-1)
    y = (jnp.dot(msg.astype(jnp.bfloat16), wp_ref[...].astype(jnp.bfloat16),
                 preferred_element_type=jnp.float32) + bp_ref[...])
    r = x_ref[...] + y
    mu = jnp.mean(r, axis=-1, keepdims=True)
    var = jnp.mean((r - mu) ** 2, axis=-1, keepdims=True)
    out_ref[...] = (r - mu) / jnp.sqrt(var + 1e-5) * g_ref[...] + b_ref[...]


def _tc_final(query, vq, msg, den, wp, bp, ln_g, ln_b, h, dph, blk=512):
    nq, dim = query.shape
    grid = (nq // blk,)
    return pl.pallas_call(
        functools.partial(_tc_final_body, h=h, dph=dph),
        grid=grid,
        in_specs=[
            pl.BlockSpec((blk, dim), lambda i: (i, 0)),
            pl.BlockSpec((blk, dim), lambda i: (i, 0)),
            pl.BlockSpec((h, blk, dph), lambda i: (0, i, 0)),
            pl.BlockSpec((2, blk, 16), lambda i: (0, i, 0)),
            pl.BlockSpec((dim, dim), lambda i: (0, 0)),
            pl.BlockSpec((1, dim), lambda i: (0, 0)),
            pl.BlockSpec((1, dim), lambda i: (0, 0)),
            pl.BlockSpec((1, dim), lambda i: (0, 0)),
        ],
        out_specs=pl.BlockSpec((blk, dim), lambda i: (i, 0)),
        out_shape=jax.ShapeDtypeStruct((nq, dim), jnp.float32),
    )(query, vq, msg, den, wp, bp.reshape(1, dim), ln_g.reshape(1, dim),
      ln_b.reshape(1, dim))


def kernel(query, keys, values, query_len, self_attn, query_idxs, key_idxs,
           value_idxs, Wq, bq, Wk, bk, Wv, bv, Wp, bp, a, prelu_w, ln_g,
           ln_b):
    k0 = keys[0]
    v0 = values[0]
    qi = query_idxs[0].astype(jnp.int32)
    ki = key_idxs[0].astype(jnp.int32)
    vi = value_idxs[0].astype(jnp.int32)

    nq, dim = query.shape
    h = a.shape[1]
    dph = a.shape[2]

    amat = jnp.zeros((dim, 16), jnp.float32)
    for hh in range(h):
        amat = amat.at[hh * dph:(hh + 1) * dph, hh].set(a[0, hh])

    qq, vq, aqq = _tc_qproj(query, Wq, bq, Wk, bk, Wv, bv, amat, prelu_w)
    perm32 = jnp.concatenate(
        [jnp.arange(0, 32, 2), jnp.arange(1, 32, 2)])
    perm = (jnp.arange(dim) // 32) * 32 + perm32[jnp.arange(dim) % 32]
    kall, vhead = _tc_kv(k0, v0, Wk, bk, Wv[:, perm], bv[perm], h, dph)
    apw = jnp.concatenate(
        [a.reshape(dim), jnp.full((LANES,), prelu_w, jnp.float32)])
    wexp, den = _sc_stage1(qi, ki, qq, kall, aqq, apw, h, dph)
    msg = _sc_stage2(qi, vi, wexp, vhead, nq, h, dph)
    return _tc_final(query, vq, msg, den, Wp, bp, ln_g, ln_b, h, dph)
